# baseline (speedup 1.0000x reference)
import functools

import jax
import jax.numpy as jnp
from jax import lax
from jax.experimental import pallas as pl
from jax.experimental.pallas import tpu as pltpu
from jax.experimental.pallas import tpu_sc as plsc

B = 2
L = 4096
D = 1024
R = 4
NH = 64
NB = 2 * NH
C = 64
CH = 512
NCH = L // CH
RB = R * B

NW = 32
ROWS_PER_W = (RB * L) // NW
SC_CH = 32
SC_NCH = ROWS_PER_W // SC_CH


def _hash_rank_body(x_ref, rm_ref, grank_ref, sh_ref, hist_s, hash_s, pb_s):
    b = pl.program_id(0)
    ch = pl.program_id(1)

    @pl.when(ch == 0)
    def _():
        hist_s[...] = jnp.zeros((R, NB), jnp.float32)

    xb = x_ref[0]
    ss = jnp.sum(xb * xb, axis=1, keepdims=True)
    xn = xb / jnp.maximum(jnp.sqrt(ss), 1e-12)
    rot = lax.dot_general(xn, rm_ref[...], (((1,), (1,)), ((), ())),
                          preferred_element_type=jnp.float32)

    iota_nh = lax.broadcasted_iota(jnp.int32, (CH, NH), 1).astype(jnp.float32)
    iota_nb = lax.broadcasted_iota(jnp.int32, (CH, NB), 1).astype(
        jnp.float32) + 1.0
    row_i = lax.broadcasted_iota(jnp.int32, (CH, CH), 0)
    col_i = lax.broadcasted_iota(jnp.int32, (CH, CH), 1)
    tri = (col_i < row_i).astype(jnp.float32)

    for r in range(R):
        rr = rot[:, r * NH:(r + 1) * NH]
        m1 = jnp.max(rr, axis=1, keepdims=True)
        m2 = jnp.min(rr, axis=1, keepdims=True)
        a1 = jnp.min(jnp.where(rr == m1, iota_nh, float(NH)), axis=1,
                     keepdims=True)
        a2 = jnp.min(jnp.where(rr == m2, iota_nh, float(NH)), axis=1,
                     keepdims=True)
        h = jnp.where(m1 >= -m2, a1 + 1.0, a2 + 1.0 + NH)
        oh = (h == iota_nb).astype(jnp.float32)
        base_cnt = hist_s[r]
        term_a = jnp.sum(oh * base_cnt[None, :], axis=1)
        within = lax.dot_general(tri, oh, (((1,), (0,)), ((), ())),
                                 preferred_element_type=jnp.float32)
        term_b = jnp.sum(within * oh, axis=1)
        pb_s[r, pl.ds(ch * CH, CH)] = term_a + term_b
        hash_s[r, pl.ds(ch * CH, CH)] = h[:, 0]
        hist_s[r] = base_cnt + jnp.sum(oh, axis=0)

    @pl.when(ch == NCH - 1)
    def _():
        row_b = lax.broadcasted_iota(jnp.int32, (NB, NB), 0)
        col_b = lax.broadcasted_iota(jnp.int32, (NB, NB), 1)
        tri_incl = (col_b <= row_b).astype(jnp.float32)
        iota_l_nb = lax.broadcasted_iota(jnp.int32, (L, NB), 1).astype(
            jnp.float32) + 1.0
        pos_l = lax.broadcasted_iota(jnp.int32, (L, NB), 0).astype(jnp.float32)
        for r in range(R):
            cnt = hist_s[r]
            incl = jnp.sum(tri_incl * cnt[None, :], axis=1)
            excl = incl - cnt
            hh = hash_s[r]
            ohf = (hh[:, None] == iota_l_nb).astype(jnp.float32)
            term_a = jnp.sum(ohf * excl[None, :], axis=1)
            rank = pb_s[r] + term_a
            grank_ref[0, r] = rank.astype(jnp.int32) + (r * B + b) * L
            sh = 1.0 + jnp.sum((incl[None, :] <= pos_l).astype(jnp.float32),
                               axis=1)
            sh_ref[0, r] = sh.astype(jnp.int32)


def _hash_rank(x, rm):
    return pl.pallas_call(
        _hash_rank_body,
        grid=(B, NCH),
        in_specs=[
            pl.BlockSpec((1, CH, D), lambda b, ch: (b, ch, 0)),
            pl.BlockSpec((R * NH, D), lambda b, ch: (0, 0)),
        ],
        out_specs=[
            pl.BlockSpec((1, R, L), lambda b, ch: (b, 0, 0)),
            pl.BlockSpec((1, R, L), lambda b, ch: (b, 0, 0)),
        ],
        out_shape=[
            jax.ShapeDtypeStruct((B, R, L), jnp.int32),
            jax.ShapeDtypeStruct((B, R, L), jnp.int32),
        ],
        scratch_shapes=[
            pltpu.VMEM((R, NB), jnp.float32),
            pltpu.VMEM((R, L), jnp.float32),
            pltpu.VMEM((R, L), jnp.float32),
        ],
    )(x, rm)


def _make_sc_scatter():
    mesh = plsc.VectorSubcoreMesh(core_axis_name="c", subcore_axis_name="s")

    @functools.partial(
        pl.kernel,
        mesh=mesh,
        out_type=[
            jax.ShapeDtypeStruct((RB * L, D), jnp.float32),
            jax.ShapeDtypeStruct((RB * L,), jnp.int32),
        ],
        scratch_types=[
            pltpu.VMEM((SC_NCH, SC_CH), jnp.int32),
            pltpu.VMEM((SC_NCH, SC_CH), jnp.int32),
            pltpu.VMEM((SC_CH, D), jnp.float32),
            pltpu.SemaphoreType.DMA,
            pltpu.SemaphoreType.DMA,
        ],
    )
    def sc_scatter(x_hbm, grank_hbm, pos_hbm, xs_hbm, si_hbm,
                   idx_v, pos_v, buf_v, sem_x, sem_i):
        w = lax.axis_index("s") * 2 + lax.axis_index("c")
        row0 = SC_NCH * w
        pltpu.sync_copy(grank_hbm.at[pl.ds(row0, SC_NCH)], idx_v)
        pltpu.sync_copy(pos_hbm.at[pl.ds(row0, SC_NCH)], pos_v)
        xbase = (w // 16) * L + (w % 4) * ROWS_PER_W

        def body(j, carry):
            pltpu.sync_copy(x_hbm.at[pl.ds(xbase + j * SC_CH, SC_CH)], buf_v)
            pltpu.async_copy(buf_v, xs_hbm.at[idx_v.at[j]], sem_x).wait()
            pltpu.async_copy(pos_v.at[j], si_hbm.at[idx_v.at[j]], sem_i).wait()
            return carry

        lax.fori_loop(0, SC_NCH, body, 0)

    return sc_scatter


def kernel(x, random_matrix):
    rm = random_matrix[:R].reshape(R * NH, D)
    grank, sh = _hash_rank(x, rm)
    grank2d = grank.reshape((RB * L) // SC_CH, SC_CH)
    pos2d = jnp.tile(jnp.arange(L, dtype=jnp.int32),
                     RB).reshape((RB * L) // SC_CH, SC_CH)
    xs_flat, si_flat = _make_sc_scatter()(x.reshape(B * L, D), grank2d, pos2d)
    x_sorted = xs_flat.reshape(R, B, L // C, C, D)
    sorted_hashes = jnp.transpose(sh, (1, 0, 2)).reshape(R, B, L // C, C)
    sorted_indices = si_flat.reshape(R, B, L, 1)
    return (x_sorted, sorted_hashes, sorted_indices, 0)


# speedup vs baseline: 3.1993x; 3.1993x over previous
import functools

import jax
import jax.numpy as jnp
from jax import lax
from jax.experimental import pallas as pl
from jax.experimental.pallas import tpu as pltpu
from jax.experimental.pallas import tpu_sc as plsc

B = 2
L = 4096
D = 1024
R = 4
NH = 64
NB = 2 * NH
C = 64
CH = 512
NCH = L // CH
RB = R * B

NW = 32
ROWS_PER_W = (RB * L) // NW
SC_CH = 32
SC_NCH = ROWS_PER_W // SC_CH


def _hash_rank_body(x_ref, rmt_ref, grank_ref, sh_ref, hist_s, hash_s, pb_s):
    b = pl.program_id(0)
    ch = pl.program_id(1)

    @pl.when(ch == 0)
    def _():
        hist_s[...] = jnp.zeros((R, NB), jnp.float32)

    xb = x_ref[0]
    ss = jnp.sum(xb * xb, axis=1, keepdims=True)
    xn = xb / jnp.maximum(jnp.sqrt(ss), 1e-12)
    rot = lax.dot_general(xn, rmt_ref[...], (((1,), (0,)), ((), ())),
                          preferred_element_type=jnp.float32)

    iota_nh = lax.broadcasted_iota(jnp.int32, (CH, NH), 1).astype(jnp.float32)
    iota_nb = lax.broadcasted_iota(jnp.int32, (CH, NB), 1).astype(
        jnp.float32) + 1.0
    row_i = lax.broadcasted_iota(jnp.int32, (CH, CH), 0)
    col_i = lax.broadcasted_iota(jnp.int32, (CH, CH), 1)
    tri = (col_i < row_i).astype(jnp.float32)

    for r in range(R):
        rr = rot[:, r * NH:(r + 1) * NH]
        m1 = jnp.max(rr, axis=1, keepdims=True)
        m2 = jnp.min(rr, axis=1, keepdims=True)
        a1 = jnp.min(jnp.where(rr == m1, iota_nh, float(NH)), axis=1,
                     keepdims=True)
        a2 = jnp.min(jnp.where(rr == m2, iota_nh, float(NH)), axis=1,
                     keepdims=True)
        h = jnp.where(m1 >= -m2, a1 + 1.0, a2 + 1.0 + NH)
        oh = (h == iota_nb).astype(jnp.float32)
        base_cnt = hist_s[r]
        term_a = jnp.sum(oh * base_cnt[None, :], axis=1, keepdims=True)
        within = lax.dot_general(tri, oh, (((1,), (0,)), ((), ())),
                                 preferred_element_type=jnp.float32)
        term_b = jnp.sum(within * oh, axis=1, keepdims=True)
        off = pl.multiple_of(ch * CH, CH)
        pb_s[r, pl.ds(off, CH)] = (term_a + term_b)[:, 0]
        hash_s[r, pl.ds(off, CH)] = h[:, 0]
        hist_s[r] = base_cnt + jnp.sum(oh, axis=0)

    @pl.when(ch == NCH - 1)
    def _():
        iota_h_col = lax.broadcasted_iota(jnp.int32, (NB, L), 0).astype(
            jnp.float32) + 1.0
        iota_pos = lax.broadcasted_iota(jnp.int32, (NB, L), 1).astype(
            jnp.float32)
        row_b = lax.broadcasted_iota(jnp.int32, (NB, NB), 0)
        col_b = lax.broadcasted_iota(jnp.int32, (NB, NB), 1)
        tri_incl = (col_b <= row_b).astype(jnp.float32)
        for r in range(R):
            cnt_col = hist_s[r][:, None]
            incl_col = lax.dot_general(tri_incl, cnt_col,
                                       (((1,), (0,)), ((), ())),
                                       preferred_element_type=jnp.float32)
            excl_col = incl_col - cnt_col
            hh = hash_s[r]
            ohf_t = (hh[None, :] == iota_h_col).astype(jnp.float32)
            term_a = jnp.sum(ohf_t * excl_col, axis=0)
            rank = pb_s[r] + term_a
            grank_ref[0, r] = rank.astype(jnp.int32) + (r * B + b) * L
            sh = 1.0 + jnp.sum((incl_col <= iota_pos).astype(jnp.float32),
                               axis=0)
            sh_ref[0, r] = sh.astype(jnp.int32)


def _hash_rank(x, rmt):
    return pl.pallas_call(
        _hash_rank_body,
        grid=(B, NCH),
        in_specs=[
            pl.BlockSpec((1, CH, D), lambda b, ch: (b, ch, 0)),
            pl.BlockSpec((D, R * NH), lambda b, ch: (0, 0)),
        ],
        out_specs=[
            pl.BlockSpec((1, R, L), lambda b, ch: (b, 0, 0)),
            pl.BlockSpec((1, R, L), lambda b, ch: (b, 0, 0)),
        ],
        out_shape=[
            jax.ShapeDtypeStruct((B, R, L), jnp.int32),
            jax.ShapeDtypeStruct((B, R, L), jnp.int32),
        ],
        scratch_shapes=[
            pltpu.VMEM((R, NB), jnp.float32),
            pltpu.VMEM((R, L), jnp.float32),
            pltpu.VMEM((R, L), jnp.float32),
        ],
    )(x, rmt)


def _make_sc_scatter():
    mesh = plsc.VectorSubcoreMesh(core_axis_name="c", subcore_axis_name="s")

    @functools.partial(
        pl.kernel,
        mesh=mesh,
        out_type=[
            jax.ShapeDtypeStruct((RB * L, D), jnp.float32),
            jax.ShapeDtypeStruct((RB * L,), jnp.int32),
        ],
        scratch_types=[
            pltpu.VMEM((SC_NCH, SC_CH), jnp.int32),
            pltpu.VMEM((SC_NCH, SC_CH), jnp.int32),
            pltpu.VMEM((SC_CH, D), jnp.float32),
            pltpu.SemaphoreType.DMA,
            pltpu.SemaphoreType.DMA,
        ],
    )
    def sc_scatter(x_hbm, grank_hbm, pos_hbm, xs_hbm, si_hbm,
                   idx_v, pos_v, buf_v, sem_x, sem_i):
        w = lax.axis_index("s") * 2 + lax.axis_index("c")
        row0 = SC_NCH * w
        pltpu.sync_copy(grank_hbm.at[pl.ds(row0, SC_NCH)], idx_v)
        pltpu.sync_copy(pos_hbm.at[pl.ds(row0, SC_NCH)], pos_v)
        xbase = (w // 16) * L + (w % 4) * ROWS_PER_W

        def body(j, carry):
            pltpu.sync_copy(x_hbm.at[pl.ds(xbase + j * SC_CH, SC_CH)], buf_v)
            pltpu.async_copy(buf_v, xs_hbm.at[idx_v.at[j]], sem_x).wait()
            pltpu.async_copy(pos_v.at[j], si_hbm.at[idx_v.at[j]], sem_i).wait()
            return carry

        lax.fori_loop(0, SC_NCH, body, 0)

    return sc_scatter


def kernel(x, random_matrix):
    rmt = random_matrix[:R].reshape(R * NH, D).T
    grank, sh = _hash_rank(x, rmt)
    grank2d = grank.reshape((RB * L) // SC_CH, SC_CH)
    pos2d = jnp.tile(jnp.arange(L, dtype=jnp.int32),
                     RB).reshape((RB * L) // SC_CH, SC_CH)
    xs_flat, si_flat = _make_sc_scatter()(x.reshape(B * L, D), grank2d, pos2d)
    x_sorted = xs_flat.reshape(R, B, L // C, C, D)
    sorted_hashes = jnp.transpose(sh, (1, 0, 2)).reshape(R, B, L // C, C)
    sorted_indices = si_flat.reshape(R, B, L, 1)
    return (x_sorted, sorted_hashes, sorted_indices, 0)
